# baseline (speedup 1.0000x reference)
import functools

import jax
import jax.numpy as jnp
from jax import lax
from jax.experimental import pallas as pl
from jax.experimental.pallas import tpu as pltpu
from jax.experimental.pallas import tpu_sc as plsc

N = 10000
E = 320000
D_IN = 128
H = 8
F = 16
HID = H * F
OUT = 64

NC = 2
NS = 16
NW = NC * NS
EW = E // NW
C = 40
NCHUNK = EW // C
NPAD = 10240
RPT = NPAD // NS

_mesh = plsc.VectorSubcoreMesh(
    core_axis_name="c", subcore_axis_name="s", num_cores=NC, num_subcores=NS)
_sc_params = pltpu.CompilerParams(
    use_tc_tiling_on_sc=False, needs_layout_passes=False)


def _wid_base():
    c = lax.axis_index("c")
    s = lax.axis_index("s")
    wid = s * NC + c
    return c, s, wid * EW


def _lanes(i):
    lanes = lax.iota(jnp.int32, 16) + 16 * i
    return lax.shift_right_logical(lanes, 3), lax.bitwise_and(lanes, 7)


@functools.partial(
    pl.kernel,
    out_type=(jax.ShapeDtypeStruct((NC, NPAD, HID), jnp.float32),
              jax.ShapeDtypeStruct((NC, NPAD, H), jnp.float32)),
    mesh=_mesh,
    compiler_params=_sc_params,
    scratch_types=(
        [pltpu.VMEM((EW,), jnp.int32),
         pltpu.VMEM((NCHUNK, C), jnp.int32)]
        + [pltpu.VMEM((C, H), jnp.float32),
           pltpu.VMEM((C, H), jnp.float32),
           pltpu.VMEM((C, HID), jnp.float32),
           pltpu.VMEM((C, H), jnp.float32),
           pltpu.SemaphoreType.DMA,
           pltpu.SemaphoreType.DMA] * 3
        + [pltpu.VMEM_SHARED((NPAD, HID), jnp.float32),
           pltpu.VMEM_SHARED((NPAD, H), jnp.float32)]
    ),
)
def _sc_edge(src_hbm, dst3_hbm, es_hbm, ed_hbm, h_hbm, z8_hbm, z128_hbm,
             outp_hbm, denp_hbm,
             sidx_all, didx_all,
             sbufA, dbufA, hbufA, exbufA, gsemA, ssemA,
             sbufB, dbufB, hbufB, exbufB, gsemB, ssemB,
             sbufC, dbufC, hbufC, exbufC, gsemC, ssemC,
             out_sh, den_sh):
    c, s, base = _wid_base()
    wid = s * NC + c
    r0 = s * RPT
    sets = ((sbufA, dbufA, hbufA, exbufA, gsemA, ssemA),
            (sbufB, dbufB, hbufB, exbufB, gsemB, ssemB),
            (sbufC, dbufC, hbufC, exbufC, gsemC, ssemC))

    def _slices(j):
        return sidx_all.at[pl.ds(j * C, C)], didx_all.at[j]

    def fire(j, S):
        sbuf, dbuf, hbuf, _, gsem, _ = S
        sl, dl = _slices(j)
        pltpu.async_copy(es_hbm.at[sl], sbuf, gsem)
        pltpu.async_copy(ed_hbm.at[dl], dbuf, gsem)
        pltpu.async_copy(h_hbm.at[sl], hbuf, gsem)

    def wait_gather(j, S):
        sbuf, dbuf, hbuf, _, gsem, _ = S
        sl, dl = _slices(j)
        pltpu.make_async_copy(es_hbm.at[sl], sbuf, gsem).wait()
        pltpu.make_async_copy(ed_hbm.at[dl], dbuf, gsem).wait()
        pltpu.make_async_copy(h_hbm.at[sl], hbuf, gsem).wait()

    def scat(j, S):
        _, _, hbuf, exbuf, _, ssem = S
        _, dl = _slices(j)
        pltpu.async_copy(hbuf, out_sh.at[dl], ssem)
        pltpu.async_copy(exbuf, den_sh.at[dl], ssem)

    def wait_scat(S):
        _, _, hbuf, exbuf, _, ssem = S
        dl = didx_all.at[0]
        pltpu.make_async_copy(hbuf, out_sh.at[dl], ssem).wait()
        pltpu.make_async_copy(exbuf, den_sh.at[dl], ssem).wait()

    def compute(j, S):
        sbuf, dbuf, hbuf, exbuf, _, _ = S
        nv = C * H // 16
        exs = []
        for i in range(nv):
            ri, ci = _lanes(i)
            e = plsc.load_gather(sbuf, [ri, ci]) + plsc.load_gather(dbuf, [ri, ci])
            ex = jnp.exp(jnp.maximum(e, 0.2 * e))
            plsc.store_scatter(exbuf, [ri, ci], ex)
            exs.append(ex)
        for i in range(nv):
            ex = exs[i]
            for half in range(2):
                eidx = 2 * i + half
                for g in range(H):
                    gidx = jnp.full((16,), half * H + g, jnp.int32)
                    a = jnp.take_along_axis(ex, gidx, axis=0,
                                            mode="promise_in_bounds")
                    hv = hbuf[eidx, pl.ds(g * F, F)]
                    hbuf[eidx, pl.ds(g * F, F)] = hv * a

    pltpu.sync_copy(src_hbm.at[pl.ds(base, EW)], sidx_all)
    pltpu.sync_copy(dst3_hbm.at[wid], didx_all)
    pltpu.sync_copy(z128_hbm, hbufA)
    pltpu.sync_copy(z8_hbm, exbufA)
    for t in range(RPT // C):
        pltpu.sync_copy(hbufA, out_sh.at[pl.ds(r0 + t * C, C), :])
        pltpu.sync_copy(exbufA, den_sh.at[pl.ds(r0 + t * C, C), :])
    plsc.subcore_barrier()

    fire(0, sets[0])
    fire(1, sets[1])
    wait_gather(0, sets[0])
    compute(0, sets[0])
    fire(2, sets[2])
    scat(0, sets[0])

    def triple(jj, carry):
        for r in range(3):
            j = 3 * jj + 1 + r
            S = sets[(1 + r) % 3]
            S2 = sets[r]
            wait_gather(j, S)
            compute(j, S)
            wait_scat(S2)
            fire(jnp.minimum(j + 2, NCHUNK - 1), S2)
            scat(j, S)
        return carry

    lax.fori_loop(0, (NCHUNK - 1) // 3, triple, 0)
    wait_gather(NCHUNK - 1, sets[1])
    wait_gather(NCHUNK - 1, sets[2])
    wait_scat(sets[0])

    plsc.subcore_barrier()
    for t in range(RPT // C):
        rr = r0 + t * C
        pltpu.sync_copy(out_sh.at[pl.ds(rr, C), :], hbufA)
        pltpu.sync_copy(hbufA, outp_hbm.at[c, pl.ds(rr, C), :])
        pltpu.sync_copy(den_sh.at[pl.ds(rr, C), :], exbufA)
        pltpu.sync_copy(exbufA, denp_hbm.at[c, pl.ds(rr, C), :])


BR = 2000
GRID = N // BR


def _combine(outp, denp):
    o = outp[0] + outp[1]
    d = denp[0] + denp[1]
    rd = 1.0 / (d + 1e-16)
    rd128 = jnp.repeat(rd, F, axis=1)
    return o * rd128


def _tc_encode_body(x_ref, w_ref, a_ref, h_ref, ee_ref):
    h = jnp.dot(x_ref[...], w_ref[...], preferred_element_type=jnp.float32)
    h_ref[...] = h
    ee_ref[...] = jnp.dot(h, a_ref[...], preferred_element_type=jnp.float32)


def _tc_post0_body(outp_ref, denp_ref, w_ref, a_ref, x_ref, h_ref, ee_ref):
    x = jax.nn.relu(_combine(outp_ref[...], denp_ref[...]))
    x_ref[...] = x
    h = jnp.dot(x, w_ref[...], preferred_element_type=jnp.float32)
    h_ref[...] = h
    ee_ref[...] = jnp.dot(h, a_ref[...], preferred_element_type=jnp.float32)


def _layer_norm(t, g, b):
    mu = jnp.mean(t, axis=-1, keepdims=True)
    var = jnp.mean((t - mu) ** 2, axis=-1, keepdims=True)
    return (t - mu) / jnp.sqrt(var + 1e-5) * g + b


def _tc_postl_body(outp_ref, denp_ref, xp_ref, g_ref, b_ref, w_ref, a_ref,
                   x_ref, h_ref, ee_ref):
    t = _combine(outp_ref[...], denp_ref[...])
    t = _layer_norm(t, g_ref[...][None, :], b_ref[...][None, :])
    x = jax.nn.relu(t) + xp_ref[...]
    x_ref[...] = x
    h = jnp.dot(x, w_ref[...], preferred_element_type=jnp.float32)
    h_ref[...] = h
    ee_ref[...] = jnp.dot(h, a_ref[...], preferred_element_type=jnp.float32)


def _tc_final_body(outp_ref, denp_ref, xp_ref, g_ref, b_ref, wp_ref, bp_ref,
                   pre_ref):
    t = _combine(outp_ref[...], denp_ref[...])
    t = _layer_norm(t, g_ref[...][None, :], b_ref[...][None, :])
    x = jax.nn.relu(t) + xp_ref[...]
    pre_ref[...] = (jnp.dot(x, wp_ref[...], preferred_element_type=jnp.float32)
                    + bp_ref[...][None, :])


_f32 = jnp.float32

_row = lambda *shape: pl.BlockSpec(shape, lambda i: (i,) + (0,) * (len(shape) - 1))
_rep = lambda *shape: pl.BlockSpec(shape, lambda i: (0,) * len(shape))
_p_spec = pl.BlockSpec((2, BR, HID), lambda i: (0, i, 0))
_d_spec = pl.BlockSpec((2, BR, H), lambda i: (0, i, 0))

_tc_encode = pl.pallas_call(
    _tc_encode_body,
    grid=(GRID,),
    in_specs=[_row(BR, D_IN), _rep(D_IN, HID), _rep(HID, 2 * H)],
    out_specs=(_row(BR, HID), _row(BR, 2 * H)),
    out_shape=(jax.ShapeDtypeStruct((N, HID), _f32),
               jax.ShapeDtypeStruct((N, 2 * H), _f32)))
_tc_post0 = pl.pallas_call(
    _tc_post0_body,
    grid=(GRID,),
    in_specs=[_p_spec, _d_spec, _rep(HID, HID), _rep(HID, 2 * H)],
    out_specs=(_row(BR, HID), _row(BR, HID), _row(BR, 2 * H)),
    out_shape=(jax.ShapeDtypeStruct((N, HID), _f32),
               jax.ShapeDtypeStruct((N, HID), _f32),
               jax.ShapeDtypeStruct((N, 2 * H), _f32)))
_tc_postl = pl.pallas_call(
    _tc_postl_body,
    grid=(GRID,),
    in_specs=[_p_spec, _d_spec, _row(BR, HID), _rep(HID), _rep(HID),
              _rep(HID, HID), _rep(HID, 2 * H)],
    out_specs=(_row(BR, HID), _row(BR, HID), _row(BR, 2 * H)),
    out_shape=(jax.ShapeDtypeStruct((N, HID), _f32),
               jax.ShapeDtypeStruct((N, HID), _f32),
               jax.ShapeDtypeStruct((N, 2 * H), _f32)))
_tc_final = pl.pallas_call(
    _tc_final_body,
    grid=(GRID,),
    in_specs=[_p_spec, _d_spec, _row(BR, HID), _rep(HID), _rep(HID),
              _rep(HID, OUT), _rep(OUT)],
    out_specs=_row(BR, OUT),
    out_shape=jax.ShapeDtypeStruct((N, OUT), _f32))


def _mk_attn(a_s, a_d):
    eye = jnp.eye(H, dtype=_f32)
    As = (a_s[:, :, None] * eye[:, None, :]).reshape(HID, H)
    Ad = (a_d[:, :, None] * eye[:, None, :]).reshape(HID, H)
    return jnp.concatenate([As, Ad], axis=1)


def kernel(nfeat, edge_index, W0, a0s, a0d, W1, a1s, a1d, W2, a2s, a2d,
           g1, b1, g2, b2, Wp, bp):
    src = edge_index[0]
    dst = edge_index[1]
    dst3 = dst.reshape(NW, NCHUNK, C)
    z8 = jnp.zeros((C, H), _f32)
    z128 = jnp.zeros((C, HID), _f32)

    h0, ee0 = _tc_encode(nfeat, W0, _mk_attn(a0s, a0d))
    outp0, denp0 = _sc_edge(src, dst3, ee0[:, :H], ee0[:, H:], h0, z8, z128)

    x1, h1, ee1 = _tc_post0(outp0, denp0, W1, _mk_attn(a1s, a1d))
    outp1, denp1 = _sc_edge(src, dst3, ee1[:, :H], ee1[:, H:], h1, z8, z128)

    x2, h2, ee2 = _tc_postl(outp1, denp1, x1, g1, b1, W2, _mk_attn(a2s, a2d))
    outp2, denp2 = _sc_edge(src, dst3, ee2[:, :H], ee2[:, H:], h2, z8, z128)

    return _tc_final(outp2, denp2, x2, g2, b2, Wp, bp)


# speedup vs baseline: 1.0784x; 1.0784x over previous
import functools

import jax
import jax.numpy as jnp
from jax import lax
from jax.experimental import pallas as pl
from jax.experimental.pallas import tpu as pltpu
from jax.experimental.pallas import tpu_sc as plsc

N = 10000
E = 320000
D_IN = 128
H = 8
F = 16
HID = H * F
OUT = 64

NC = 2
NS = 16
NW = NC * NS
EW = E // NW
C = 40
NCHUNK = EW // C
NPAD = 10240
RPT = NPAD // NS

_mesh = plsc.VectorSubcoreMesh(
    core_axis_name="c", subcore_axis_name="s", num_cores=NC, num_subcores=NS)
_sc_params = pltpu.CompilerParams(
    use_tc_tiling_on_sc=False, needs_layout_passes=False)


def _wid_base():
    c = lax.axis_index("c")
    s = lax.axis_index("s")
    wid = s * NC + c
    return c, s, wid * EW


def _lanes(i):
    lanes = lax.iota(jnp.int32, 16) + 16 * i
    return lax.shift_right_logical(lanes, 3), lax.bitwise_and(lanes, 7)


@functools.partial(
    pl.kernel,
    out_type=(jax.ShapeDtypeStruct((NC, NPAD, HID), jnp.float32),
              jax.ShapeDtypeStruct((NC, NPAD, H), jnp.float32)),
    mesh=_mesh,
    compiler_params=_sc_params,
    scratch_types=[
        pltpu.VMEM((EW,), jnp.int32),
        pltpu.VMEM((NCHUNK, C), jnp.int32),
        pltpu.VMEM((C, H), jnp.float32),
        pltpu.VMEM((C, H), jnp.float32),
        pltpu.VMEM((C, HID), jnp.float32),
        pltpu.VMEM((C, H), jnp.float32),
        pltpu.VMEM((C, H), jnp.float32),
        pltpu.VMEM((C, HID), jnp.float32),
        pltpu.VMEM((C, H), jnp.float32),
        pltpu.SemaphoreType.DMA,
        pltpu.SemaphoreType.DMA,
        pltpu.VMEM_SHARED((NPAD, HID), jnp.float32),
        pltpu.VMEM_SHARED((NPAD, H), jnp.float32),
    ],
)
def _sc_edge(src_hbm, dst3_hbm, es_hbm, ed_hbm, h_hbm, z8_hbm, z128_hbm,
             outp_hbm, denp_hbm,
             sidx_all, didx_all, sbufA, dbufA, hbufA, sbufB, dbufB, hbufB,
             exbuf, semA, semB, out_sh, den_sh):
    c, s, base = _wid_base()
    wid = s * NC + c
    r0 = s * RPT

    def _slices(j):
        return sidx_all.at[pl.ds(j * C, C)], didx_all.at[j]

    def fire(j, sbuf, dbuf, hbuf, sem):
        sl, dl = _slices(j)
        pltpu.async_copy(es_hbm.at[sl], sbuf, sem)
        pltpu.async_copy(ed_hbm.at[dl], dbuf, sem)
        pltpu.async_copy(h_hbm.at[sl], hbuf, sem)

    def wait(j, sbuf, dbuf, hbuf, sem):
        sl, dl = _slices(j)
        pltpu.make_async_copy(es_hbm.at[sl], sbuf, sem).wait()
        pltpu.make_async_copy(ed_hbm.at[dl], dbuf, sem).wait()
        pltpu.make_async_copy(h_hbm.at[sl], hbuf, sem).wait()

    def compute_scatter(j, sbuf, dbuf, hbuf):
        _, dl = _slices(j)
        nv = C * H // 16
        exs = []
        for i in range(nv):
            ri, ci = _lanes(i)
            e = plsc.load_gather(sbuf, [ri, ci]) + plsc.load_gather(dbuf, [ri, ci])
            ex = jnp.exp(jnp.maximum(e, 0.2 * e))
            plsc.store_scatter(exbuf, [ri, ci], ex)
            exs.append(ex)
        for i in range(nv):
            ex = exs[i]
            for half in range(2):
                eidx = 2 * i + half
                for g in range(H):
                    gidx = jnp.full((16,), half * H + g, jnp.int32)
                    a = jnp.take_along_axis(ex, gidx, axis=0,
                                            mode="promise_in_bounds")
                    hv = hbuf[eidx, pl.ds(g * F, F)]
                    hbuf[eidx, pl.ds(g * F, F)] = hv * a
        pltpu.sync_copy(hbuf, out_sh.at[dl], add=True)
        pltpu.sync_copy(exbuf, den_sh.at[dl], add=True)

    pltpu.sync_copy(src_hbm.at[pl.ds(base, EW)], sidx_all)
    pltpu.sync_copy(dst3_hbm.at[wid], didx_all)
    pltpu.sync_copy(z128_hbm, hbufA)
    pltpu.sync_copy(z8_hbm, sbufA)
    for t in range(RPT // C):
        pltpu.sync_copy(hbufA, out_sh.at[pl.ds(r0 + t * C, C), :])
        pltpu.sync_copy(sbufA, den_sh.at[pl.ds(r0 + t * C, C), :])
    plsc.subcore_barrier()

    fire(0, sbufA, dbufA, hbufA, semA)

    def pair(jj, carry):
        j0 = 2 * jj
        fire(j0 + 1, sbufB, dbufB, hbufB, semB)
        wait(j0, sbufA, dbufA, hbufA, semA)
        compute_scatter(j0, sbufA, dbufA, hbufA)
        fire(j0 + 2, sbufA, dbufA, hbufA, semA)
        wait(j0 + 1, sbufB, dbufB, hbufB, semB)
        compute_scatter(j0 + 1, sbufB, dbufB, hbufB)
        return carry

    lax.fori_loop(0, NCHUNK // 2 - 1, pair, 0)
    jl = NCHUNK - 2
    fire(jl + 1, sbufB, dbufB, hbufB, semB)
    wait(jl, sbufA, dbufA, hbufA, semA)
    compute_scatter(jl, sbufA, dbufA, hbufA)
    wait(jl + 1, sbufB, dbufB, hbufB, semB)
    compute_scatter(jl + 1, sbufB, dbufB, hbufB)

    plsc.subcore_barrier()
    for t in range(RPT // C):
        rr = r0 + t * C
        pltpu.sync_copy(out_sh.at[pl.ds(rr, C), :], hbufA)
        pltpu.sync_copy(hbufA, outp_hbm.at[c, pl.ds(rr, C), :])
        pltpu.sync_copy(den_sh.at[pl.ds(rr, C), :], sbufA)
        pltpu.sync_copy(sbufA, denp_hbm.at[c, pl.ds(rr, C), :])


BR = 2000
GRID = N // BR


def _combine(outp, denp):
    o = outp[0] + outp[1]
    d = denp[0] + denp[1]
    rd = 1.0 / (d + 1e-16)
    rd128 = jnp.repeat(rd, F, axis=1)
    return o * rd128


def _emit(h, a, h_ref, es_ref, ed_ref):
    h_ref[...] = h
    ee = jnp.dot(h, a, preferred_element_type=jnp.float32)
    es_ref[...] = ee[:, :H]
    ed_ref[...] = ee[:, H:]


def _tc_encode_body(x_ref, w_ref, a_ref, h_ref, es_ref, ed_ref):
    h = jnp.dot(x_ref[...], w_ref[...], preferred_element_type=jnp.float32)
    _emit(h, a_ref[...], h_ref, es_ref, ed_ref)


def _tc_post0_body(outp_ref, denp_ref, w_ref, a_ref, x_ref, h_ref,
                   es_ref, ed_ref):
    x = jax.nn.relu(_combine(outp_ref[...], denp_ref[...]))
    x_ref[...] = x
    h = jnp.dot(x, w_ref[...], preferred_element_type=jnp.float32)
    _emit(h, a_ref[...], h_ref, es_ref, ed_ref)


def _layer_norm(t, g, b):
    mu = jnp.mean(t, axis=-1, keepdims=True)
    var = jnp.mean((t - mu) ** 2, axis=-1, keepdims=True)
    return (t - mu) / jnp.sqrt(var + 1e-5) * g + b


def _tc_postl_body(outp_ref, denp_ref, xp_ref, g_ref, b_ref, w_ref, a_ref,
                   x_ref, h_ref, es_ref, ed_ref):
    t = _combine(outp_ref[...], denp_ref[...])
    t = _layer_norm(t, g_ref[...][None, :], b_ref[...][None, :])
    x = jax.nn.relu(t) + xp_ref[...]
    x_ref[...] = x
    h = jnp.dot(x, w_ref[...], preferred_element_type=jnp.float32)
    _emit(h, a_ref[...], h_ref, es_ref, ed_ref)


def _tc_final_body(outp_ref, denp_ref, xp_ref, g_ref, b_ref, wp_ref, bp_ref,
                   pre_ref):
    t = _combine(outp_ref[...], denp_ref[...])
    t = _layer_norm(t, g_ref[...][None, :], b_ref[...][None, :])
    x = jax.nn.relu(t) + xp_ref[...]
    pre_ref[...] = (jnp.dot(x, wp_ref[...], preferred_element_type=jnp.float32)
                    + bp_ref[...][None, :])


_f32 = jnp.float32

_row = lambda *shape: pl.BlockSpec(shape, lambda i: (i,) + (0,) * (len(shape) - 1))
_rep = lambda *shape: pl.BlockSpec(shape, lambda i: (0,) * len(shape))
_p_spec = pl.BlockSpec((2, BR, HID), lambda i: (0, i, 0))
_d_spec = pl.BlockSpec((2, BR, H), lambda i: (0, i, 0))

_tc_encode = pl.pallas_call(
    _tc_encode_body,
    grid=(GRID,),
    in_specs=[_row(BR, D_IN), _rep(D_IN, HID), _rep(HID, 2 * H)],
    out_specs=(_row(BR, HID), _row(BR, H), _row(BR, H)),
    out_shape=(jax.ShapeDtypeStruct((N, HID), _f32),
               jax.ShapeDtypeStruct((N, H), _f32),
               jax.ShapeDtypeStruct((N, H), _f32)))
_tc_post0 = pl.pallas_call(
    _tc_post0_body,
    grid=(GRID,),
    in_specs=[_p_spec, _d_spec, _rep(HID, HID), _rep(HID, 2 * H)],
    out_specs=(_row(BR, HID), _row(BR, HID), _row(BR, H), _row(BR, H)),
    out_shape=(jax.ShapeDtypeStruct((N, HID), _f32),
               jax.ShapeDtypeStruct((N, HID), _f32),
               jax.ShapeDtypeStruct((N, H), _f32),
               jax.ShapeDtypeStruct((N, H), _f32)))
_tc_postl = pl.pallas_call(
    _tc_postl_body,
    grid=(GRID,),
    in_specs=[_p_spec, _d_spec, _row(BR, HID), _rep(HID), _rep(HID),
              _rep(HID, HID), _rep(HID, 2 * H)],
    out_specs=(_row(BR, HID), _row(BR, HID), _row(BR, H), _row(BR, H)),
    out_shape=(jax.ShapeDtypeStruct((N, HID), _f32),
               jax.ShapeDtypeStruct((N, HID), _f32),
               jax.ShapeDtypeStruct((N, H), _f32),
               jax.ShapeDtypeStruct((N, H), _f32)))
_tc_final = pl.pallas_call(
    _tc_final_body,
    grid=(GRID,),
    in_specs=[_p_spec, _d_spec, _row(BR, HID), _rep(HID), _rep(HID),
              _rep(HID, OUT), _rep(OUT)],
    out_specs=_row(BR, OUT),
    out_shape=jax.ShapeDtypeStruct((N, OUT), _f32))


def _mk_attn(a_s, a_d):
    eye = jnp.eye(H, dtype=_f32)
    As = (a_s[:, :, None] * eye[:, None, :]).reshape(HID, H)
    Ad = (a_d[:, :, None] * eye[:, None, :]).reshape(HID, H)
    return jnp.concatenate([As, Ad], axis=1)


def kernel(nfeat, edge_index, W0, a0s, a0d, W1, a1s, a1d, W2, a2s, a2d,
           g1, b1, g2, b2, Wp, bp):
    src = edge_index[0]
    dst = edge_index[1]
    dst3 = dst.reshape(NW, NCHUNK, C)
    z8 = jnp.zeros((C, H), _f32)
    z128 = jnp.zeros((C, HID), _f32)

    h0, es0, ed0 = _tc_encode(nfeat, W0, _mk_attn(a0s, a0d))
    outp0, denp0 = _sc_edge(src, dst3, es0, ed0, h0, z8, z128)

    x1, h1, es1, ed1 = _tc_post0(outp0, denp0, W1, _mk_attn(a1s, a1d))
    outp1, denp1 = _sc_edge(src, dst3, es1, ed1, h1, z8, z128)

    x2, h2, es2, ed2 = _tc_postl(outp1, denp1, x1, g1, b1, W2, _mk_attn(a2s, a2d))
    outp2, denp2 = _sc_edge(src, dst3, es2, ed2, h2, z8, z128)

    return _tc_final(outp2, denp2, x2, g2, b2, Wp, bp)


# speedup vs baseline: 1.1353x; 1.0528x over previous
import functools

import jax
import jax.numpy as jnp
from jax import lax
from jax.experimental import pallas as pl
from jax.experimental.pallas import tpu as pltpu
from jax.experimental.pallas import tpu_sc as plsc

N = 10000
E = 320000
D_IN = 128
H = 8
F = 16
HID = H * F
OUT = 64

NC = 2
NS = 16
NW = NC * NS
EW = E // NW
C = 40
NCHUNK = EW // C
NPAD = 10240
RPT = NPAD // NS

_mesh = plsc.VectorSubcoreMesh(
    core_axis_name="c", subcore_axis_name="s", num_cores=NC, num_subcores=NS)
_sc_params = pltpu.CompilerParams(
    use_tc_tiling_on_sc=False, needs_layout_passes=False)


def _wid_base():
    c = lax.axis_index("c")
    s = lax.axis_index("s")
    wid = s * NC + c
    return c, s, wid * EW


def _lanes(i):
    lanes = lax.iota(jnp.int32, 16) + 16 * i
    return lax.shift_right_logical(lanes, 3), lax.bitwise_and(lanes, 7)


@functools.partial(
    pl.kernel,
    out_type=(jax.ShapeDtypeStruct((NC, NPAD, HID), jnp.float32),
              jax.ShapeDtypeStruct((NC, NPAD, H), jnp.float32)),
    mesh=_mesh,
    compiler_params=_sc_params,
    scratch_types=[
        pltpu.VMEM((EW,), jnp.int32),
        pltpu.VMEM((NCHUNK, C), jnp.int32),
        pltpu.VMEM((C, H), jnp.float32),
        pltpu.VMEM((C, H), jnp.float32),
        pltpu.VMEM((C, HID), jnp.float32),
        pltpu.VMEM((C, H), jnp.float32),
        pltpu.VMEM((C, H), jnp.float32),
        pltpu.VMEM((C, HID), jnp.float32),
        pltpu.VMEM((C, H), jnp.float32),
        pltpu.SemaphoreType.DMA,
        pltpu.SemaphoreType.DMA,
        pltpu.VMEM_SHARED((NPAD, HID), jnp.float32),
        pltpu.VMEM_SHARED((NPAD, H), jnp.float32),
    ],
)
def _sc_edge(src_hbm, dst3_hbm, es_hbm, ed_hbm, h_hbm, z8_hbm, z128_hbm,
             outp_hbm, denp_hbm,
             sidx_all, didx_all, sbufA, dbufA, hbufA, sbufB, dbufB, hbufB,
             exbuf, semA, semB, out_sh, den_sh):
    c, s, base = _wid_base()
    wid = s * NC + c
    r0 = s * RPT

    def _slices(j):
        return sidx_all.at[pl.ds(j * C, C)], didx_all.at[j]

    def fire(j, sbuf, dbuf, hbuf, sem):
        sl, dl = _slices(j)
        pltpu.async_copy(es_hbm.at[sl], sbuf, sem)
        pltpu.async_copy(ed_hbm.at[dl], dbuf, sem)
        pltpu.async_copy(h_hbm.at[sl], hbuf, sem)

    def wait(j, sbuf, dbuf, hbuf, sem):
        sl, dl = _slices(j)
        pltpu.make_async_copy(es_hbm.at[sl], sbuf, sem).wait()
        pltpu.make_async_copy(ed_hbm.at[dl], dbuf, sem).wait()
        pltpu.make_async_copy(h_hbm.at[sl], hbuf, sem).wait()

    def compute_scatter(j, sbuf, dbuf, hbuf):
        _, dl = _slices(j)
        nv = C * H // 16
        exs = []
        for i in range(nv):
            ri, ci = _lanes(i)
            e = plsc.load_gather(sbuf, [ri, ci]) + plsc.load_gather(dbuf, [ri, ci])
            ex = jnp.exp(jnp.maximum(e, 0.2 * e))
            plsc.store_scatter(exbuf, [ri, ci], ex)
            exs.append(ex)
        for i in range(nv):
            ex = exs[i]
            for half in range(2):
                eidx = 2 * i + half
                for g in range(H):
                    gidx = jnp.full((16,), half * H + g, jnp.int32)
                    a = jnp.take_along_axis(ex, gidx, axis=0,
                                            mode="promise_in_bounds")
                    hv = hbuf[eidx, pl.ds(g * F, F)]
                    hbuf[eidx, pl.ds(g * F, F)] = hv * a
        pltpu.sync_copy(hbuf, out_sh.at[dl], add=True)
        pltpu.sync_copy(exbuf, den_sh.at[dl], add=True)

    pltpu.sync_copy(src_hbm.at[pl.ds(base, EW)], sidx_all)
    pltpu.sync_copy(dst3_hbm.at[wid], didx_all)
    pltpu.sync_copy(z128_hbm, hbufA)
    pltpu.sync_copy(z8_hbm, sbufA)
    for t in range(RPT // C):
        pltpu.sync_copy(hbufA, out_sh.at[pl.ds(r0 + t * C, C), :])
        pltpu.sync_copy(sbufA, den_sh.at[pl.ds(r0 + t * C, C), :])
    plsc.subcore_barrier()

    fire(0, sbufA, dbufA, hbufA, semA)

    def pair(jj, carry):
        j0 = 2 * jj
        fire(j0 + 1, sbufB, dbufB, hbufB, semB)
        wait(j0, sbufA, dbufA, hbufA, semA)
        compute_scatter(j0, sbufA, dbufA, hbufA)
        fire(j0 + 2, sbufA, dbufA, hbufA, semA)
        wait(j0 + 1, sbufB, dbufB, hbufB, semB)
        compute_scatter(j0 + 1, sbufB, dbufB, hbufB)
        return carry

    lax.fori_loop(0, NCHUNK // 2 - 1, pair, 0)
    jl = NCHUNK - 2
    fire(jl + 1, sbufB, dbufB, hbufB, semB)
    wait(jl, sbufA, dbufA, hbufA, semA)
    compute_scatter(jl, sbufA, dbufA, hbufA)
    wait(jl + 1, sbufB, dbufB, hbufB, semB)
    compute_scatter(jl + 1, sbufB, dbufB, hbufB)

    plsc.subcore_barrier()
    for t in range(RPT // C):
        rr = r0 + t * C
        pltpu.sync_copy(out_sh.at[pl.ds(rr, C), :], hbufA)
        pltpu.sync_copy(hbufA, outp_hbm.at[c, pl.ds(rr, C), :])
        pltpu.sync_copy(den_sh.at[pl.ds(rr, C), :], sbufA)
        pltpu.sync_copy(sbufA, denp_hbm.at[c, pl.ds(rr, C), :])


BR = 2000
GRID = N // BR


def _combine(outp, denp):
    o = outp[0] + outp[1]
    d = denp[0] + denp[1]
    rd = 1.0 / (d + 1e-16)
    row = lax.broadcasted_iota(jnp.int32, (H, HID), 0)
    col = lax.broadcasted_iota(jnp.int32, (H, HID), 1)
    expand = (lax.div(col, F) == row).astype(jnp.float32)
    rd128 = jnp.dot(rd, expand, preferred_element_type=jnp.float32)
    return o * rd128


def _emit(h, a, h_ref, es_ref, ed_ref):
    h_ref[...] = h
    ee = jnp.dot(h, a, preferred_element_type=jnp.float32)
    es_ref[...] = ee[:, :H]
    ed_ref[...] = ee[:, H:]


def _tc_encode_body(x_ref, w_ref, a_ref, h_ref, es_ref, ed_ref):
    h = jnp.dot(x_ref[...], w_ref[...], preferred_element_type=jnp.float32)
    _emit(h, a_ref[...], h_ref, es_ref, ed_ref)


def _tc_post0_body(outp_ref, denp_ref, w_ref, a_ref, x_ref, h_ref,
                   es_ref, ed_ref):
    x = jax.nn.relu(_combine(outp_ref[...], denp_ref[...]))
    x_ref[...] = x
    h = jnp.dot(x, w_ref[...], preferred_element_type=jnp.float32)
    _emit(h, a_ref[...], h_ref, es_ref, ed_ref)


def _layer_norm(t, g, b):
    mu = jnp.mean(t, axis=-1, keepdims=True)
    var = jnp.mean((t - mu) ** 2, axis=-1, keepdims=True)
    return (t - mu) / jnp.sqrt(var + 1e-5) * g + b


def _tc_postl_body(outp_ref, denp_ref, xp_ref, g_ref, b_ref, w_ref, a_ref,
                   x_ref, h_ref, es_ref, ed_ref):
    t = _combine(outp_ref[...], denp_ref[...])
    t = _layer_norm(t, g_ref[...][None, :], b_ref[...][None, :])
    x = jax.nn.relu(t) + xp_ref[...]
    x_ref[...] = x
    h = jnp.dot(x, w_ref[...], preferred_element_type=jnp.float32)
    _emit(h, a_ref[...], h_ref, es_ref, ed_ref)


def _tc_final_body(outp_ref, denp_ref, xp_ref, g_ref, b_ref, wp_ref, bp_ref,
                   pre_ref):
    t = _combine(outp_ref[...], denp_ref[...])
    t = _layer_norm(t, g_ref[...][None, :], b_ref[...][None, :])
    x = jax.nn.relu(t) + xp_ref[...]
    pre_ref[...] = (jnp.dot(x, wp_ref[...], preferred_element_type=jnp.float32)
                    + bp_ref[...][None, :])


_f32 = jnp.float32

_row = lambda *shape: pl.BlockSpec(shape, lambda i: (i,) + (0,) * (len(shape) - 1))
_rep = lambda *shape: pl.BlockSpec(shape, lambda i: (0,) * len(shape))
_p_spec = pl.BlockSpec((2, BR, HID), lambda i: (0, i, 0))
_d_spec = pl.BlockSpec((2, BR, H), lambda i: (0, i, 0))

_tc_encode = pl.pallas_call(
    _tc_encode_body,
    grid=(GRID,),
    in_specs=[_row(BR, D_IN), _rep(D_IN, HID), _rep(HID, 2 * H)],
    out_specs=(_row(BR, HID), _row(BR, H), _row(BR, H)),
    out_shape=(jax.ShapeDtypeStruct((N, HID), _f32),
               jax.ShapeDtypeStruct((N, H), _f32),
               jax.ShapeDtypeStruct((N, H), _f32)))
_tc_post0 = pl.pallas_call(
    _tc_post0_body,
    grid=(GRID,),
    in_specs=[_p_spec, _d_spec, _rep(HID, HID), _rep(HID, 2 * H)],
    out_specs=(_row(BR, HID), _row(BR, HID), _row(BR, H), _row(BR, H)),
    out_shape=(jax.ShapeDtypeStruct((N, HID), _f32),
               jax.ShapeDtypeStruct((N, HID), _f32),
               jax.ShapeDtypeStruct((N, H), _f32),
               jax.ShapeDtypeStruct((N, H), _f32)))
_tc_postl = pl.pallas_call(
    _tc_postl_body,
    grid=(GRID,),
    in_specs=[_p_spec, _d_spec, _row(BR, HID), _rep(HID), _rep(HID),
              _rep(HID, HID), _rep(HID, 2 * H)],
    out_specs=(_row(BR, HID), _row(BR, HID), _row(BR, H), _row(BR, H)),
    out_shape=(jax.ShapeDtypeStruct((N, HID), _f32),
               jax.ShapeDtypeStruct((N, HID), _f32),
               jax.ShapeDtypeStruct((N, H), _f32),
               jax.ShapeDtypeStruct((N, H), _f32)))
_tc_final = pl.pallas_call(
    _tc_final_body,
    grid=(GRID,),
    in_specs=[_p_spec, _d_spec, _row(BR, HID), _rep(HID), _rep(HID),
              _rep(HID, OUT), _rep(OUT)],
    out_specs=_row(BR, OUT),
    out_shape=jax.ShapeDtypeStruct((N, OUT), _f32))


def _mk_attn(a_s, a_d):
    eye = jnp.eye(H, dtype=_f32)
    As = (a_s[:, :, None] * eye[:, None, :]).reshape(HID, H)
    Ad = (a_d[:, :, None] * eye[:, None, :]).reshape(HID, H)
    return jnp.concatenate([As, Ad], axis=1)


def kernel(nfeat, edge_index, W0, a0s, a0d, W1, a1s, a1d, W2, a2s, a2d,
           g1, b1, g2, b2, Wp, bp):
    src = edge_index[0]
    dst = edge_index[1]
    dst3 = dst.reshape(NW, NCHUNK, C)
    z8 = jnp.zeros((C, H), _f32)
    z128 = jnp.zeros((C, HID), _f32)

    h0, es0, ed0 = _tc_encode(nfeat, W0, _mk_attn(a0s, a0d))
    outp0, denp0 = _sc_edge(src, dst3, es0, ed0, h0, z8, z128)

    x1, h1, es1, ed1 = _tc_post0(outp0, denp0, W1, _mk_attn(a1s, a1d))
    outp1, denp1 = _sc_edge(src, dst3, es1, ed1, h1, z8, z128)

    x2, h2, es2, ed2 = _tc_postl(outp1, denp1, x1, g1, b1, W2, _mk_attn(a2s, a2d))
    outp2, denp2 = _sc_edge(src, dst3, es2, ed2, h2, z8, z128)

    return _tc_final(outp2, denp2, x2, g2, b2, Wp, bp)
